# baseline (speedup 1.0000x reference)
import jax
import jax.numpy as jnp
from jax import lax
from jax.experimental import pallas as pl
from jax.experimental.pallas import tpu as pltpu
from jax.experimental.pallas import tpu_sc as plsc

N_NODES = 10000
N_PAD = 10240
E_EDGES = 320000
D_IN = 128
D_OUT = 128
N_WORKERS = 32
CHUNK = 80
N_CHUNKS = 125
ROWS_PER_TILE = N_PAD // 16


def _mm_body(x_ref, wb_ref, wl_ref, b_ref, yb_ref, yl_ref):
    x = x_ref[...]
    y = jnp.dot(x, wb_ref[...], preferred_element_type=jnp.float32)
    b0 = jax.lax.bitcast_convert_type(
        y[:, :D_OUT].astype(jnp.bfloat16), jnp.uint16).astype(jnp.uint32)
    b1 = jax.lax.bitcast_convert_type(
        y[:, D_OUT:].astype(jnp.bfloat16), jnp.uint16).astype(jnp.uint32)
    yb_ref[...] = jax.lax.bitcast_convert_type(
        b0 | (b1 << 16), jnp.float32)
    yl_ref[...] = (
        jnp.dot(x, wl_ref[...], preferred_element_type=jnp.float32) + b_ref[...]
    )


def _final_body(p0_ref, p1_ref, yl_ref, o_ref):
    o_ref[...] = p0_ref[...] + p1_ref[...] + yl_ref[...]


def _sc_body(yb_hbm, src_hbm, dst_hbm, et_hbm, c0t_hbm, c1t_hbm,
             p0_hbm, p1_hbm,
             srcv, dstv, etv, rows, mbuf, c0tab, c1tab, acc, sem):
    cid = lax.axis_index("c")
    sid = lax.axis_index("s")
    wid = cid * 16 + sid

    pltpu.sync_copy(c0t_hbm, c0tab)
    pltpu.sync_copy(c1t_hbm, c1tab)
    c0t_vec = c0tab[...]
    c1t_vec = c1tab[...]
    s0 = [c0t_vec[r] for r in range(8)]
    s1 = [c1t_vec[r] for r in range(8)]

    def _zrow(r, carry):
        for j in range(8):
            mbuf[r, pl.ds(j * 16, 16)] = jnp.zeros((16,), jnp.float32)
        return carry
    lax.fori_loop(0, CHUNK, _zrow, 0)
    nfull = ROWS_PER_TILE // CHUNK
    for k in range(nfull):
        pltpu.sync_copy(mbuf, acc.at[pl.ds(sid * ROWS_PER_TILE + k * CHUNK, CHUNK)])
    rem = ROWS_PER_TILE - nfull * CHUNK
    if rem:
        pltpu.sync_copy(mbuf.at[pl.ds(0, rem)],
                        acc.at[pl.ds(sid * ROWS_PER_TILE + nfull * CHUNK, rem)])
    plsc.subcore_barrier()

    def _gather(slot_b, _c):
        pltpu.async_copy(yb_hbm.at[srcv.at[slot_b]], rows.at[slot_b], sem)

    def _idx_issue(slot_b, chunk_c):
        pltpu.async_copy(src_hbm.at[wid, chunk_c], srcv.at[slot_b], sem)
        pltpu.async_copy(dst_hbm.at[wid, chunk_c], dstv.at[slot_b], sem)
        pltpu.async_copy(et_hbm.at[wid, chunk_c], etv.at[slot_b], sem)

    def _idx_wait(slot_b):
        pltpu.make_async_copy(src_hbm.at[wid, 0], srcv.at[slot_b], sem).wait()
        pltpu.make_async_copy(dst_hbm.at[wid, 0], dstv.at[slot_b], sem).wait()
        pltpu.make_async_copy(et_hbm.at[wid, 0], etv.at[slot_b], sem).wait()

    def _gather_wait(slot_b):
        pltpu.make_async_copy(yb_hbm.at[srcv.at[slot_b]], rows.at[slot_b],
                              sem).wait()

    def _compute(slot_b):
        for g in range(CHUNK // 16):
            et = etv[slot_b, pl.ds(g * 16, 16)]
            c0 = jnp.zeros((16,), jnp.float32)
            c1 = jnp.zeros((16,), jnp.float32)
            for r in range(8):
                msk = et == r
                c0 = jnp.where(msk, s0[r], c0)
                c1 = jnp.where(msk, s1[r], c1)
            for t in range(16):
                e = g * 16 + t
                c0s = c0[t]
                c1s = c1[t]
                for j in range(8):
                    w = rows[slot_b, e, pl.ds(j * 16, 16)]
                    wi = jax.lax.bitcast_convert_type(w, jnp.uint32)
                    y0 = jax.lax.bitcast_convert_type(wi << 16, jnp.float32)
                    y1 = jax.lax.bitcast_convert_type(
                        wi & jnp.uint32(0xFFFF0000), jnp.float32)
                    mbuf[e, pl.ds(j * 16, 16)] = c0s * y0 + c1s * y1

    pltpu.sync_copy(src_hbm.at[wid, 0], srcv.at[0])
    pltpu.sync_copy(dst_hbm.at[wid, 0], dstv.at[0])
    pltpu.sync_copy(et_hbm.at[wid, 0], etv.at[0])
    _gather(0, 0)
    _idx_issue(1, 1)

    def _chunk(c, carry):
        b = lax.rem(c, 2)
        nb = 1 - b
        _gather_wait(b)
        _idx_wait(nb)
        _gather(nb, c + 1)
        _compute(b)
        pltpu.sync_copy(mbuf, acc.at[dstv.at[b]], add=True)
        c_next = jnp.minimum(c + 2, N_CHUNKS - 1)
        _idx_issue(b, c_next)
        return carry
    lax.fori_loop(0, N_CHUNKS - 1, _chunk, 0)

    last_b = (N_CHUNKS - 1) % 2
    _gather_wait(last_b)
    _compute(last_b)
    pltpu.sync_copy(mbuf, acc.at[dstv.at[last_b]], add=True)
    _idx_wait(1 - last_b)
    plsc.subcore_barrier()

    row0 = sid * ROWS_PER_TILE

    @pl.when(cid == 0)
    def _():
        pltpu.sync_copy(acc.at[pl.ds(row0, ROWS_PER_TILE)],
                        p0_hbm.at[pl.ds(row0, ROWS_PER_TILE)])

    @pl.when(cid == 1)
    def _():
        pltpu.sync_copy(acc.at[pl.ds(row0, ROWS_PER_TILE)],
                        p1_hbm.at[pl.ds(row0, ROWS_PER_TILE)])


_sc_call = pl.kernel(
    _sc_body,
    out_type=[jax.ShapeDtypeStruct((N_PAD, D_OUT), jnp.float32)] * 2,
    mesh=plsc.VectorSubcoreMesh(core_axis_name="c", subcore_axis_name="s"),
    scratch_types=[
        pltpu.VMEM((2, CHUNK), jnp.int32),
        pltpu.VMEM((2, CHUNK), jnp.int32),
        pltpu.VMEM((2, CHUNK), jnp.int32),
        pltpu.VMEM((2, CHUNK, D_IN), jnp.float32),
        pltpu.VMEM((CHUNK, D_OUT), jnp.float32),
        pltpu.VMEM((16,), jnp.float32),
        pltpu.VMEM((16,), jnp.float32),
        pltpu.VMEM_SHARED((N_PAD, D_OUT), jnp.float32),
        pltpu.SemaphoreType.DMA,
    ],
)


@jax.jit
def kernel(feat, edge_index, etypes, W, coeff, h_bias, loop_weight):
    feat_p = jnp.zeros((N_PAD, D_IN), jnp.float32).at[:N_NODES].set(feat)
    wb = jnp.concatenate([W[0], W[1]], axis=1)
    bias2d = h_bias.reshape(1, D_OUT)

    grid = N_PAD // 512
    yb, yl = pl.pallas_call(
        _mm_body,
        grid=(grid,),
        in_specs=[
            pl.BlockSpec((512, D_IN), lambda i: (i, 0)),
            pl.BlockSpec((D_IN, 2 * D_OUT), lambda i: (0, 0)),
            pl.BlockSpec((D_IN, D_OUT), lambda i: (0, 0)),
            pl.BlockSpec((1, D_OUT), lambda i: (0, 0)),
        ],
        out_specs=[
            pl.BlockSpec((512, D_OUT), lambda i: (i, 0)),
            pl.BlockSpec((512, D_OUT), lambda i: (i, 0)),
        ],
        out_shape=[
            jax.ShapeDtypeStruct((N_PAD, D_OUT), jnp.float32),
            jax.ShapeDtypeStruct((N_PAD, D_OUT), jnp.float32),
        ],
    )(feat_p, wb, loop_weight, bias2d)

    src3 = edge_index[0].reshape(N_WORKERS, N_CHUNKS, CHUNK)
    dst3 = edge_index[1].reshape(N_WORKERS, N_CHUNKS, CHUNK)
    et3 = etypes.reshape(N_WORKERS, N_CHUNKS, CHUNK)
    c0t = jnp.zeros((16,), jnp.float32).at[:coeff.shape[0]].set(coeff[:, 0])
    c1t = jnp.zeros((16,), jnp.float32).at[:coeff.shape[0]].set(coeff[:, 1])

    p0, p1 = _sc_call(yb, src3, dst3, et3, c0t, c1t)

    out = pl.pallas_call(
        _final_body,
        grid=(grid,),
        in_specs=[pl.BlockSpec((512, D_OUT), lambda i: (i, 0))] * 3,
        out_specs=pl.BlockSpec((512, D_OUT), lambda i: (i, 0)),
        out_shape=jax.ShapeDtypeStruct((N_PAD, D_OUT), jnp.float32),
    )(p0, p1, yl)
    return out[:N_NODES]


# speedup vs baseline: 1.3952x; 1.3952x over previous
import jax
import jax.numpy as jnp
from jax import lax
from jax.experimental import pallas as pl
from jax.experimental.pallas import tpu as pltpu
from jax.experimental.pallas import tpu_sc as plsc

N_NODES = 10000
N_PAD = 10240
E_EDGES = 320000
D_IN = 128
D_OUT = 128
N_WORKERS = 32
CHUNK = 80
N_CHUNKS = 125
ROWS_PER_TILE = N_PAD // 16


def _mm_body(x_ref, wb_ref, wl_ref, b_ref, yb_ref, yl_ref):
    x = x_ref[...]
    y = jnp.dot(x, wb_ref[...], preferred_element_type=jnp.float32)
    b0 = jax.lax.bitcast_convert_type(
        y[:, :D_OUT].astype(jnp.bfloat16), jnp.uint16).astype(jnp.uint32)
    b1 = jax.lax.bitcast_convert_type(
        y[:, D_OUT:].astype(jnp.bfloat16), jnp.uint16).astype(jnp.uint32)
    yb_ref[...] = jax.lax.bitcast_convert_type(
        b0 | (b1 << 16), jnp.float32)
    yl_ref[...] = (
        jnp.dot(x, wl_ref[...], preferred_element_type=jnp.float32) + b_ref[...]
    )


def _final_body(p0_ref, p1_ref, yl_ref, o_ref):
    o_ref[...] = p0_ref[...] + p1_ref[...] + yl_ref[...]


def _sc_body(yb_hbm, src_hbm, dst_hbm, et_hbm, c0t_hbm, c1t_hbm,
             p0_hbm, p1_hbm,
             srcv, dstv, etv, rows, mbuf, c0tab, c1tab, acc, sem):
    cid = lax.axis_index("c")
    sid = lax.axis_index("s")
    wid = cid * 16 + sid

    pltpu.sync_copy(c0t_hbm, c0tab)
    pltpu.sync_copy(c1t_hbm, c1tab)
    c0t_vec = c0tab[...]
    c1t_vec = c1tab[...]
    s0 = [c0t_vec[r] for r in range(8)]
    s1 = [c1t_vec[r] for r in range(8)]

    def _zrow(r, carry):
        for j in range(8):
            mbuf[r, pl.ds(j * 16, 16)] = jnp.zeros((16,), jnp.float32)
        return carry
    lax.fori_loop(0, CHUNK, _zrow, 0)
    nfull = ROWS_PER_TILE // CHUNK
    for k in range(nfull):
        pltpu.sync_copy(mbuf, acc.at[pl.ds(sid * ROWS_PER_TILE + k * CHUNK, CHUNK)])
    rem = ROWS_PER_TILE - nfull * CHUNK
    if rem:
        pltpu.sync_copy(mbuf.at[pl.ds(0, rem)],
                        acc.at[pl.ds(sid * ROWS_PER_TILE + nfull * CHUNK, rem)])
    plsc.subcore_barrier()

    def _gather(slot_b, _c):
        pltpu.async_copy(yb_hbm.at[srcv.at[slot_b]], rows.at[slot_b], sem)

    def _idx_issue(slot_b, chunk_c):
        pltpu.async_copy(src_hbm.at[wid, chunk_c], srcv.at[slot_b], sem)
        pltpu.async_copy(dst_hbm.at[wid, chunk_c], dstv.at[slot_b], sem)
        pltpu.async_copy(et_hbm.at[wid, chunk_c], etv.at[slot_b], sem)

    def _idx_wait(slot_b):
        pltpu.make_async_copy(src_hbm.at[wid, 0], srcv.at[slot_b], sem).wait()
        pltpu.make_async_copy(dst_hbm.at[wid, 0], dstv.at[slot_b], sem).wait()
        pltpu.make_async_copy(et_hbm.at[wid, 0], etv.at[slot_b], sem).wait()

    def _gather_wait(slot_b):
        pltpu.make_async_copy(yb_hbm.at[srcv.at[slot_b]], rows.at[slot_b],
                              sem).wait()

    def _compute(slot_b):
        def _group(g, carry):
            et = etv[slot_b, pl.ds(g * 16, 16)]
            c0 = jnp.zeros((16,), jnp.float32)
            c1 = jnp.zeros((16,), jnp.float32)
            for r in range(8):
                msk = et == r
                c0 = jnp.where(msk, s0[r], c0)
                c1 = jnp.where(msk, s1[r], c1)
            for t in range(16):
                e = g * 16 + t
                c0s = c0[t]
                c1s = c1[t]
                for j in range(8):
                    w = rows[slot_b, e, pl.ds(j * 16, 16)]
                    wi = jax.lax.bitcast_convert_type(w, jnp.uint32)
                    y0 = jax.lax.bitcast_convert_type(wi << 16, jnp.float32)
                    y1 = jax.lax.bitcast_convert_type(
                        wi & jnp.uint32(0xFFFF0000), jnp.float32)
                    mbuf[e, pl.ds(j * 16, 16)] = c0s * y0 + c1s * y1
            return carry
        lax.fori_loop(0, CHUNK // 16, _group, 0)

    pltpu.sync_copy(src_hbm.at[wid, 0], srcv.at[0])
    pltpu.sync_copy(dst_hbm.at[wid, 0], dstv.at[0])
    pltpu.sync_copy(et_hbm.at[wid, 0], etv.at[0])
    _gather(0, 0)
    _idx_issue(1, 1)

    def _chunk(c, carry):
        b = lax.rem(c, 2)
        nb = 1 - b
        _gather_wait(b)
        _idx_wait(nb)
        _gather(nb, c + 1)
        _compute(b)
        pltpu.sync_copy(mbuf, acc.at[dstv.at[b]], add=True)
        c_next = jnp.minimum(c + 2, N_CHUNKS - 1)
        _idx_issue(b, c_next)
        return carry
    lax.fori_loop(0, N_CHUNKS - 1, _chunk, 0)

    last_b = (N_CHUNKS - 1) % 2
    _gather_wait(last_b)
    _compute(last_b)
    pltpu.sync_copy(mbuf, acc.at[dstv.at[last_b]], add=True)
    _idx_wait(1 - last_b)
    plsc.subcore_barrier()

    row0 = sid * ROWS_PER_TILE

    @pl.when(cid == 0)
    def _():
        pltpu.sync_copy(acc.at[pl.ds(row0, ROWS_PER_TILE)],
                        p0_hbm.at[pl.ds(row0, ROWS_PER_TILE)])

    @pl.when(cid == 1)
    def _():
        pltpu.sync_copy(acc.at[pl.ds(row0, ROWS_PER_TILE)],
                        p1_hbm.at[pl.ds(row0, ROWS_PER_TILE)])


_sc_call = pl.kernel(
    _sc_body,
    out_type=[jax.ShapeDtypeStruct((N_PAD, D_OUT), jnp.float32)] * 2,
    mesh=plsc.VectorSubcoreMesh(core_axis_name="c", subcore_axis_name="s"),
    scratch_types=[
        pltpu.VMEM((2, CHUNK), jnp.int32),
        pltpu.VMEM((2, CHUNK), jnp.int32),
        pltpu.VMEM((2, CHUNK), jnp.int32),
        pltpu.VMEM((2, CHUNK, D_IN), jnp.float32),
        pltpu.VMEM((CHUNK, D_OUT), jnp.float32),
        pltpu.VMEM((16,), jnp.float32),
        pltpu.VMEM((16,), jnp.float32),
        pltpu.VMEM_SHARED((N_PAD, D_OUT), jnp.float32),
        pltpu.SemaphoreType.DMA,
    ],
)


@jax.jit
def kernel(feat, edge_index, etypes, W, coeff, h_bias, loop_weight):
    feat_p = jnp.zeros((N_PAD, D_IN), jnp.float32).at[:N_NODES].set(feat)
    wb = jnp.concatenate([W[0], W[1]], axis=1)
    bias2d = h_bias.reshape(1, D_OUT)

    grid = N_PAD // 512
    yb, yl = pl.pallas_call(
        _mm_body,
        grid=(grid,),
        in_specs=[
            pl.BlockSpec((512, D_IN), lambda i: (i, 0)),
            pl.BlockSpec((D_IN, 2 * D_OUT), lambda i: (0, 0)),
            pl.BlockSpec((D_IN, D_OUT), lambda i: (0, 0)),
            pl.BlockSpec((1, D_OUT), lambda i: (0, 0)),
        ],
        out_specs=[
            pl.BlockSpec((512, D_OUT), lambda i: (i, 0)),
            pl.BlockSpec((512, D_OUT), lambda i: (i, 0)),
        ],
        out_shape=[
            jax.ShapeDtypeStruct((N_PAD, D_OUT), jnp.float32),
            jax.ShapeDtypeStruct((N_PAD, D_OUT), jnp.float32),
        ],
    )(feat_p, wb, loop_weight, bias2d)

    src3 = edge_index[0].reshape(N_WORKERS, N_CHUNKS, CHUNK)
    dst3 = edge_index[1].reshape(N_WORKERS, N_CHUNKS, CHUNK)
    et3 = etypes.reshape(N_WORKERS, N_CHUNKS, CHUNK)
    c0t = jnp.zeros((16,), jnp.float32).at[:coeff.shape[0]].set(coeff[:, 0])
    c1t = jnp.zeros((16,), jnp.float32).at[:coeff.shape[0]].set(coeff[:, 1])

    p0, p1 = _sc_call(yb, src3, dst3, et3, c0t, c1t)

    out = pl.pallas_call(
        _final_body,
        grid=(grid,),
        in_specs=[pl.BlockSpec((512, D_OUT), lambda i: (i, 0))] * 3,
        out_specs=pl.BlockSpec((512, D_OUT), lambda i: (i, 0)),
        out_shape=jax.ShapeDtypeStruct((N_PAD, D_OUT), jnp.float32),
    )(p0, p1, yl)
    return out[:N_NODES]


# speedup vs baseline: 1.5022x; 1.0767x over previous
import jax
import jax.numpy as jnp
from jax import lax
from jax.experimental import pallas as pl
from jax.experimental.pallas import tpu as pltpu
from jax.experimental.pallas import tpu_sc as plsc

N_NODES = 10000
N_PAD = 10240
E_EDGES = 320000
D_IN = 128
D_OUT = 128
N_WORKERS = 32
CHUNK = 80
N_CHUNKS = 125
ROWS_PER_TILE = N_PAD // 16


def _mm_body(x_ref, wb_ref, wl_ref, b_ref, yb_ref, yl_ref):
    x = x_ref[...]
    y = jnp.dot(x, wb_ref[...], preferred_element_type=jnp.float32)
    b0 = jax.lax.bitcast_convert_type(
        y[:, :D_OUT].astype(jnp.bfloat16), jnp.uint16).astype(jnp.uint32)
    b1 = jax.lax.bitcast_convert_type(
        y[:, D_OUT:].astype(jnp.bfloat16), jnp.uint16).astype(jnp.uint32)
    yb_ref[...] = jax.lax.bitcast_convert_type(
        b0 | (b1 << 16), jnp.float32)
    yl_ref[...] = (
        jnp.dot(x, wl_ref[...], preferred_element_type=jnp.float32) + b_ref[...]
    )


def _final_body(p0_ref, p1_ref, yl_ref, o_ref):
    o_ref[...] = p0_ref[...] + p1_ref[...] + yl_ref[...]


def _sc_body(yb_hbm, src_hbm, dst_hbm, et_hbm, c0t_hbm, c1t_hbm,
             p0_hbm, p1_hbm,
             srcv, dstv, etv, rows, mbuf, c0tab, c1tab, acc,
             sem, sem_s0, sem_s1):
    cid = lax.axis_index("c")
    sid = lax.axis_index("s")
    wid = cid * 16 + sid

    pltpu.sync_copy(c0t_hbm, c0tab)
    pltpu.sync_copy(c1t_hbm, c1tab)
    c0t_vec = c0tab[...]
    c1t_vec = c1tab[...]
    s0 = [c0t_vec[r] for r in range(8)]
    s1 = [c1t_vec[r] for r in range(8)]

    def _zrow(r, carry):
        for j in range(8):
            mbuf[0, r, pl.ds(j * 16, 16)] = jnp.zeros((16,), jnp.float32)
            mbuf[1, r, pl.ds(j * 16, 16)] = jnp.zeros((16,), jnp.float32)
        return carry
    lax.fori_loop(0, CHUNK, _zrow, 0)
    nfull = ROWS_PER_TILE // CHUNK
    for k in range(nfull):
        pltpu.sync_copy(mbuf.at[0],
                        acc.at[pl.ds(sid * ROWS_PER_TILE + k * CHUNK, CHUNK)])
    rem = ROWS_PER_TILE - nfull * CHUNK
    if rem:
        pltpu.sync_copy(mbuf.at[0, pl.ds(0, rem)],
                        acc.at[pl.ds(sid * ROWS_PER_TILE + nfull * CHUNK, rem)])
    plsc.subcore_barrier()

    def _gather(slot_b, _c):
        pltpu.async_copy(yb_hbm.at[srcv.at[slot_b]], rows.at[slot_b], sem)

    def _idx_issue(slot_b, slot_d, chunk_c):
        pltpu.async_copy(src_hbm.at[wid, chunk_c], srcv.at[slot_b], sem)
        pltpu.async_copy(dst_hbm.at[wid, chunk_c], dstv.at[slot_d], sem)
        pltpu.async_copy(et_hbm.at[wid, chunk_c], etv.at[slot_b], sem)

    def _idx_wait(slot_b):
        pltpu.make_async_copy(src_hbm.at[wid, 0], srcv.at[slot_b], sem).wait()
        pltpu.make_async_copy(dst_hbm.at[wid, 0], dstv.at[0], sem).wait()
        pltpu.make_async_copy(et_hbm.at[wid, 0], etv.at[slot_b], sem).wait()

    def _scatter_wait(sem_s):
        pltpu.make_async_copy(mbuf.at[0],
                              acc.at[pl.ds(sid * ROWS_PER_TILE, CHUNK)],
                              sem_s).wait()

    def _gather_wait(slot_b):
        pltpu.make_async_copy(yb_hbm.at[srcv.at[slot_b]], rows.at[slot_b],
                              sem).wait()

    def _compute(slot_b):
        def _group(g, carry):
            et = etv[slot_b, pl.ds(g * 16, 16)]
            c0 = jnp.zeros((16,), jnp.float32)
            c1 = jnp.zeros((16,), jnp.float32)
            for r in range(8):
                msk = et == r
                c0 = jnp.where(msk, s0[r], c0)
                c1 = jnp.where(msk, s1[r], c1)
            for t in range(16):
                e = g * 16 + t
                c0s = c0[t]
                c1s = c1[t]
                for j in range(8):
                    w = rows[slot_b, e, pl.ds(j * 16, 16)]
                    wi = jax.lax.bitcast_convert_type(w, jnp.uint32)
                    y0 = jax.lax.bitcast_convert_type(wi << 16, jnp.float32)
                    y1 = jax.lax.bitcast_convert_type(
                        wi & jnp.uint32(0xFFFF0000), jnp.float32)
                    mbuf[slot_b, e, pl.ds(j * 16, 16)] = c0s * y0 + c1s * y1
            return carry
        lax.fori_loop(0, CHUNK // 16, _group, 0)

    pltpu.sync_copy(src_hbm.at[wid, 0], srcv.at[0])
    pltpu.sync_copy(dst_hbm.at[wid, 0], dstv.at[0])
    pltpu.sync_copy(dst_hbm.at[wid, 0], dstv.at[2])
    pltpu.sync_copy(dst_hbm.at[wid, 0], dstv.at[3])
    pltpu.sync_copy(et_hbm.at[wid, 0], etv.at[0])
    _gather(0, 0)
    _idx_issue(1, 1, 1)
    pltpu.async_copy(mbuf.at[0], acc.at[dstv.at[2]], sem_s0, add=True)
    pltpu.async_copy(mbuf.at[1], acc.at[dstv.at[3]], sem_s1, add=True)

    def _chunk(c, carry):
        b = lax.rem(c, 2)
        nb = 1 - b
        d4 = lax.rem(c, 4)
        _gather_wait(b)
        _idx_wait(nb)
        _gather(nb, c + 1)
        @pl.when(b == 0)
        def _():
            _scatter_wait(sem_s0)

        @pl.when(b == 1)
        def _():
            _scatter_wait(sem_s1)
        _compute(b)

        @pl.when(b == 0)
        def _():
            pltpu.async_copy(mbuf.at[0], acc.at[dstv.at[d4]], sem_s0,
                             add=True)

        @pl.when(b == 1)
        def _():
            pltpu.async_copy(mbuf.at[1], acc.at[dstv.at[d4]], sem_s1,
                             add=True)
        c_next = jnp.minimum(c + 2, N_CHUNKS - 1)
        _idx_issue(b, lax.rem(c + 2, 4), c_next)
        return carry
    lax.fori_loop(0, N_CHUNKS - 1, _chunk, 0)

    last_c = N_CHUNKS - 1
    last_b = last_c % 2
    _gather_wait(last_b)
    _scatter_wait(sem_s0 if last_b == 0 else sem_s1)
    _compute(last_b)
    pltpu.sync_copy(mbuf.at[last_b], acc.at[dstv.at[last_c % 4]], add=True)
    _idx_wait(1 - last_b)
    _scatter_wait(sem_s1 if last_b == 0 else sem_s0)
    plsc.subcore_barrier()

    row0 = sid * ROWS_PER_TILE

    @pl.when(cid == 0)
    def _():
        pltpu.sync_copy(acc.at[pl.ds(row0, ROWS_PER_TILE)],
                        p0_hbm.at[pl.ds(row0, ROWS_PER_TILE)])

    @pl.when(cid == 1)
    def _():
        pltpu.sync_copy(acc.at[pl.ds(row0, ROWS_PER_TILE)],
                        p1_hbm.at[pl.ds(row0, ROWS_PER_TILE)])


_sc_call = pl.kernel(
    _sc_body,
    out_type=[jax.ShapeDtypeStruct((N_PAD, D_OUT), jnp.float32)] * 2,
    mesh=plsc.VectorSubcoreMesh(core_axis_name="c", subcore_axis_name="s"),
    scratch_types=[
        pltpu.VMEM((2, CHUNK), jnp.int32),
        pltpu.VMEM((4, CHUNK), jnp.int32),
        pltpu.VMEM((2, CHUNK), jnp.int32),
        pltpu.VMEM((2, CHUNK, D_IN), jnp.float32),
        pltpu.VMEM((2, CHUNK, D_OUT), jnp.float32),
        pltpu.VMEM((16,), jnp.float32),
        pltpu.VMEM((16,), jnp.float32),
        pltpu.VMEM_SHARED((N_PAD, D_OUT), jnp.float32),
        pltpu.SemaphoreType.DMA,
        pltpu.SemaphoreType.DMA,
        pltpu.SemaphoreType.DMA,
    ],
)


@jax.jit
def kernel(feat, edge_index, etypes, W, coeff, h_bias, loop_weight):
    feat_p = jnp.zeros((N_PAD, D_IN), jnp.float32).at[:N_NODES].set(feat)
    wb = jnp.concatenate([W[0], W[1]], axis=1)
    bias2d = h_bias.reshape(1, D_OUT)

    grid = N_PAD // 512
    yb, yl = pl.pallas_call(
        _mm_body,
        grid=(grid,),
        in_specs=[
            pl.BlockSpec((512, D_IN), lambda i: (i, 0)),
            pl.BlockSpec((D_IN, 2 * D_OUT), lambda i: (0, 0)),
            pl.BlockSpec((D_IN, D_OUT), lambda i: (0, 0)),
            pl.BlockSpec((1, D_OUT), lambda i: (0, 0)),
        ],
        out_specs=[
            pl.BlockSpec((512, D_OUT), lambda i: (i, 0)),
            pl.BlockSpec((512, D_OUT), lambda i: (i, 0)),
        ],
        out_shape=[
            jax.ShapeDtypeStruct((N_PAD, D_OUT), jnp.float32),
            jax.ShapeDtypeStruct((N_PAD, D_OUT), jnp.float32),
        ],
    )(feat_p, wb, loop_weight, bias2d)

    src3 = edge_index[0].reshape(N_WORKERS, N_CHUNKS, CHUNK)
    dst3 = edge_index[1].reshape(N_WORKERS, N_CHUNKS, CHUNK)
    et3 = etypes.reshape(N_WORKERS, N_CHUNKS, CHUNK)
    c0t = jnp.zeros((16,), jnp.float32).at[:coeff.shape[0]].set(coeff[:, 0])
    c1t = jnp.zeros((16,), jnp.float32).at[:coeff.shape[0]].set(coeff[:, 1])

    p0, p1 = _sc_call(yb, src3, dst3, et3, c0t, c1t)

    out = pl.pallas_call(
        _final_body,
        grid=(grid,),
        in_specs=[pl.BlockSpec((512, D_OUT), lambda i: (i, 0))] * 3,
        out_specs=pl.BlockSpec((512, D_OUT), lambda i: (i, 0)),
        out_shape=jax.ShapeDtypeStruct((N_PAD, D_OUT), jnp.float32),
    )(p0, p1, yl)
    return out[:N_NODES]
